# baseline (device time: 31400 ns/iter reference)
import jax
import jax.numpy as jnp
from jax import lax
from jax.experimental import pallas as pl
from jax.experimental.pallas import tpu as pltpu

P = 8
ZD = (0, 1, 2)
XR = (3, 4)
YR = (5, 6, 7)


def kernel(x, pi):
    _, m, n = x.shape
    quarter = m // 4
    assert quarter % P == 0
    rows = quarter // P
    n_zsend = P + len(ZD)

    def body(x_ref, pi_ref, out_ref, zsend, xstage, ld_sems,
             z_s, z_r, xf_s, xf_r, yf_s, yf_r, xr_s, xr_r, yr_s, yr_r):
        mx = lax.axis_index("x")
        my = lax.axis_index("y")
        mz = lax.axis_index("z")

        dst_z = pi_ref[mz]
        src_z = jnp.where(pi_ref[0] == mz, 0, 1)

        q_me = 2 * mx + my
        q_x = 2 * (1 - mx) + my
        q_y = 2 * mx + (1 - my)
        q_d = 2 * (1 - mx) + (1 - my)

        def out_rows(q, p):
            return out_ref.at[0, pl.ds(q * quarter + p * rows, rows), :]

        barrier = pltpu.get_barrier_semaphore()
        for dev in [(mx, my, dst_z), (mx, my, src_z),
                    (1 - mx, my, mz), (mx, 1 - my, mz)]:
            pl.semaphore_signal(
                barrier, inc=1, device_id=dev,
                device_id_type=pl.DeviceIdType.MESH,
            )
        pl.semaphore_wait(barrier, 4)

        def rdma(src, dst, ssem, rsem, dev):
            return pltpu.make_async_remote_copy(
                src_ref=src, dst_ref=dst, send_sem=ssem, recv_sem=rsem,
                device_id=dev, device_id_type=pl.DeviceIdType.MESH,
            )

        z_dev = (mx, my, dst_z)
        x_dev = (1 - mx, my, mz)
        y_dev = (mx, 1 - my, mz)

        def z_quarter_piece(k):
            q, p = (q_me, k) if k < P else (q_d, ZD[k - P])
            return q, p

        def z_rdma(k):
            q, p = z_quarter_piece(k)
            return rdma(zsend.at[k], out_rows(q, p), z_s.at[k], z_r.at[k],
                        z_dev)

        def xf_rdma(p):
            return rdma(out_rows(q_me, p), out_rows(q_me, p), xf_s.at[p],
                        xf_r.at[p], x_dev)

        def yf_rdma(p):
            return rdma(out_rows(q_me, p), out_rows(q_me, p), yf_s.at[p],
                        yf_r.at[p], y_dev)

        def xr_rdma(j):
            p = XR[j]
            return rdma(out_rows(q_y, p), out_rows(q_y, p), xr_s.at[j],
                        xr_r.at[j], x_dev)

        def yr_rdma(j):
            p = YR[j]
            return rdma(out_rows(q_x, p), out_rows(q_x, p), yr_s.at[j],
                        yr_r.at[j], y_dev)

        for k in range(n_zsend):
            q, p = z_quarter_piece(k)
            pltpu.make_async_copy(
                x_ref.at[0, pl.ds(q * quarter + p * rows, rows), :],
                xstage.at[k],
                ld_sems.at[k],
            ).start()
        for k in range(n_zsend):
            q, p = z_quarter_piece(k)
            pltpu.make_async_copy(
                x_ref.at[0, pl.ds(q * quarter + p * rows, rows), :],
                xstage.at[k],
                ld_sems.at[k],
            ).wait()
            zsend[k, :, :] = xstage[k, :, :].astype(jnp.bfloat16)
            z_rdma(k).start()

        for p in range(P):
            z_rdma(p).wait_recv()
            xf_rdma(p).start()
            yf_rdma(p).start()

        for p in range(P):
            rdma(zsend.at[0], out_rows(q_x, p), z_s.at[0], xf_r.at[p],
                 x_dev).wait_recv()
            if p in YR:
                yr_rdma(YR.index(p)).start()
            rdma(zsend.at[0], out_rows(q_y, p), z_s.at[0], yf_r.at[p],
                 y_dev).wait_recv()
            if p in XR:
                xr_rdma(XR.index(p)).start()

        for k in range(P, n_zsend):
            z_rdma(k).wait_recv()
        for j in range(len(XR)):
            rdma(zsend.at[0], out_rows(q_d, XR[j]), z_s.at[0], xr_r.at[j],
                 x_dev).wait_recv()
        for j in range(len(YR)):
            rdma(zsend.at[0], out_rows(q_d, YR[j]), z_s.at[0], yr_r.at[j],
                 y_dev).wait_recv()

        for k in range(n_zsend):
            z_rdma(k).wait_send()
        for p in range(P):
            xf_rdma(p).wait_send()
            yf_rdma(p).wait_send()
        for j in range(len(XR)):
            xr_rdma(j).wait_send()
        for j in range(len(YR)):
            yr_rdma(j).wait_send()

    return pl.pallas_call(
        body,
        out_shape=jax.ShapeDtypeStruct(x.shape, jnp.bfloat16),
        in_specs=[
            pl.BlockSpec(memory_space=pl.ANY),
            pl.BlockSpec(memory_space=pltpu.SMEM),
        ],
        out_specs=pl.BlockSpec(memory_space=pltpu.VMEM),
        scratch_shapes=[
            pltpu.VMEM((n_zsend, rows, n), jnp.bfloat16),
            pltpu.VMEM((n_zsend, rows, n), jnp.float32),
            pltpu.SemaphoreType.DMA((n_zsend,)),
            pltpu.SemaphoreType.DMA((n_zsend,)),
            pltpu.SemaphoreType.DMA((n_zsend,)),
            pltpu.SemaphoreType.DMA((P,)),
            pltpu.SemaphoreType.DMA((P,)),
            pltpu.SemaphoreType.DMA((P,)),
            pltpu.SemaphoreType.DMA((P,)),
            pltpu.SemaphoreType.DMA((len(XR),)),
            pltpu.SemaphoreType.DMA((len(XR),)),
            pltpu.SemaphoreType.DMA((len(YR),)),
            pltpu.SemaphoreType.DMA((len(YR),)),
        ],
        compiler_params=pltpu.CompilerParams(collective_id=0),
    )(x, pi)


# device time: 30031 ns/iter; 1.0456x vs baseline; 1.0456x over previous
import jax
import jax.numpy as jnp
from jax import lax
from jax.experimental import pallas as pl
from jax.experimental.pallas import tpu as pltpu

P = 8
ZD = (0, 1, 2, 3)
XR = (4, 5)
YR = (6, 7)


def kernel(x, pi):
    _, m, n = x.shape
    quarter = m // 4
    assert quarter % P == 0
    rows = quarter // P
    n_zsend = P + len(ZD)

    def body(x_ref, pi_ref, out_ref, zsend, xstage, ld_sems,
             z_s, z_r, xf_s, xf_r, yf_s, yf_r, xr_s, xr_r, yr_s, yr_r):
        mx = lax.axis_index("x")
        my = lax.axis_index("y")
        mz = lax.axis_index("z")

        dst_z = pi_ref[mz]
        src_z = jnp.where(pi_ref[0] == mz, 0, 1)

        q_me = 2 * mx + my
        q_x = 2 * (1 - mx) + my
        q_y = 2 * mx + (1 - my)
        q_d = 2 * (1 - mx) + (1 - my)

        def out_rows(q, p):
            return out_ref.at[0, pl.ds(q * quarter + p * rows, rows), :]

        barrier = pltpu.get_barrier_semaphore()
        for dev in [(mx, my, dst_z), (mx, my, src_z),
                    (1 - mx, my, mz), (mx, 1 - my, mz)]:
            pl.semaphore_signal(
                barrier, inc=1, device_id=dev,
                device_id_type=pl.DeviceIdType.MESH,
            )
        pl.semaphore_wait(barrier, 4)

        def rdma(src, dst, ssem, rsem, dev):
            return pltpu.make_async_remote_copy(
                src_ref=src, dst_ref=dst, send_sem=ssem, recv_sem=rsem,
                device_id=dev, device_id_type=pl.DeviceIdType.MESH,
            )

        z_dev = (mx, my, dst_z)
        x_dev = (1 - mx, my, mz)
        y_dev = (mx, 1 - my, mz)

        def z_quarter_piece(k):
            q, p = (q_me, k) if k < P else (q_d, ZD[k - P])
            return q, p

        def z_rdma(k):
            q, p = z_quarter_piece(k)
            return rdma(zsend.at[k], out_rows(q, p), z_s.at[k], z_r.at[k],
                        z_dev)

        def xf_rdma(p):
            return rdma(out_rows(q_me, p), out_rows(q_me, p), xf_s.at[p],
                        xf_r.at[p], x_dev)

        def yf_rdma(p):
            return rdma(out_rows(q_me, p), out_rows(q_me, p), yf_s.at[p],
                        yf_r.at[p], y_dev)

        def xr_rdma(j):
            p = XR[j]
            return rdma(out_rows(q_y, p), out_rows(q_y, p), xr_s.at[j],
                        xr_r.at[j], x_dev)

        def yr_rdma(j):
            p = YR[j]
            return rdma(out_rows(q_x, p), out_rows(q_x, p), yr_s.at[j],
                        yr_r.at[j], y_dev)

        for k in range(n_zsend):
            q, p = z_quarter_piece(k)
            pltpu.make_async_copy(
                x_ref.at[0, pl.ds(q * quarter + p * rows, rows), :],
                xstage.at[k],
                ld_sems.at[k],
            ).start()
        for k in range(n_zsend):
            q, p = z_quarter_piece(k)
            pltpu.make_async_copy(
                x_ref.at[0, pl.ds(q * quarter + p * rows, rows), :],
                xstage.at[k],
                ld_sems.at[k],
            ).wait()
            zsend[k, :, :] = xstage[k, :, :].astype(jnp.bfloat16)
            z_rdma(k).start()

        for p in range(P):
            z_rdma(p).wait_recv()
            xf_rdma(p).start()
            yf_rdma(p).start()

        for p in range(P):
            rdma(zsend.at[0], out_rows(q_x, p), z_s.at[0], xf_r.at[p],
                 x_dev).wait_recv()
            if p in YR:
                yr_rdma(YR.index(p)).start()
            rdma(zsend.at[0], out_rows(q_y, p), z_s.at[0], yf_r.at[p],
                 y_dev).wait_recv()
            if p in XR:
                xr_rdma(XR.index(p)).start()

        for k in range(P, n_zsend):
            z_rdma(k).wait_recv()
        for j in range(len(XR)):
            rdma(zsend.at[0], out_rows(q_d, XR[j]), z_s.at[0], xr_r.at[j],
                 x_dev).wait_recv()
        for j in range(len(YR)):
            rdma(zsend.at[0], out_rows(q_d, YR[j]), z_s.at[0], yr_r.at[j],
                 y_dev).wait_recv()

        for k in range(n_zsend):
            z_rdma(k).wait_send()
        for p in range(P):
            xf_rdma(p).wait_send()
            yf_rdma(p).wait_send()
        for j in range(len(XR)):
            xr_rdma(j).wait_send()
        for j in range(len(YR)):
            yr_rdma(j).wait_send()

    return pl.pallas_call(
        body,
        out_shape=jax.ShapeDtypeStruct(x.shape, jnp.bfloat16),
        in_specs=[
            pl.BlockSpec(memory_space=pl.ANY),
            pl.BlockSpec(memory_space=pltpu.SMEM),
        ],
        out_specs=pl.BlockSpec(memory_space=pltpu.VMEM),
        scratch_shapes=[
            pltpu.VMEM((n_zsend, rows, n), jnp.bfloat16),
            pltpu.VMEM((n_zsend, rows, n), jnp.float32),
            pltpu.SemaphoreType.DMA((n_zsend,)),
            pltpu.SemaphoreType.DMA((n_zsend,)),
            pltpu.SemaphoreType.DMA((n_zsend,)),
            pltpu.SemaphoreType.DMA((P,)),
            pltpu.SemaphoreType.DMA((P,)),
            pltpu.SemaphoreType.DMA((P,)),
            pltpu.SemaphoreType.DMA((P,)),
            pltpu.SemaphoreType.DMA((len(XR),)),
            pltpu.SemaphoreType.DMA((len(XR),)),
            pltpu.SemaphoreType.DMA((len(YR),)),
            pltpu.SemaphoreType.DMA((len(YR),)),
        ],
        compiler_params=pltpu.CompilerParams(collective_id=0),
    )(x, pi)
